# Initial kernel scaffold; baseline (speedup 1.0000x reference)
import jax
import jax.numpy as jnp
from jax.experimental import pallas as pl


def kernel(x, edge_index, lin_weight, lin_bias, att, conv_bias):
    raise NotImplementedError("write your pallas kernel here")


# speedup vs baseline: 11.9922x; 11.9922x over previous
import functools
import jax
import jax.numpy as jnp
from jax.experimental import pallas as pl

C = 1.0
HEADS = 4
OUT_CH = 32
N = 10000
D = 128
MAXN = 1.0 - 4e-3
BLK = 1000


def _artanh(x):
    x = jnp.clip(x, -1 + 1e-7, 1 - 1e-7)
    return 0.5 * jnp.log((1 + x) / (1 - x))


def _rownorm(x):
    return jnp.maximum(jnp.sqrt(jnp.sum(x * x, axis=-1, keepdims=True)), 1e-15)


def _proj_rows(x):
    n = _rownorm(x)
    return jnp.where(n > MAXN, x / n * MAXN, x)


def _stage1_body(x_ref, wt_ref, b_ref, out_ref):
    x = x_ref[...]
    wt = wt_ref[...]
    b = b_ref[...]
    xn = _rownorm(x)
    mx = jnp.dot(x, wt, preferred_element_type=jnp.float32)
    mxn = _rownorm(mx)
    h = jnp.tanh(mxn / xn * _artanh(xn)) * mx / mxn
    h = _proj_rows(h)
    bn = jnp.maximum(jnp.sqrt(jnp.sum(b * b)), 1e-15)
    hb = jnp.tanh(bn) * b / bn
    hbn = jnp.maximum(jnp.sqrt(jnp.sum(hb * hb)), 1e-15)
    hb = jnp.where(hbn > MAXN, hb / hbn * MAXN, hb)
    hb2 = jnp.sum(hb * hb)
    x2 = jnp.sum(h * h, axis=-1, keepdims=True)
    xy = jnp.sum(h * hb[None, :], axis=-1, keepdims=True)
    num = (1 + 2 * xy + hb2) * h + (1 - x2) * hb[None, :]
    den = 1 + 2 * xy + x2 * hb2
    h = num / jnp.maximum(den, 1e-15)
    h = _proj_rows(h)
    hn = _rownorm(h)
    out_ref[...] = h / hn * _artanh(hn)


def _stage2_body(g_ref, att_ref, ai_ref, aj_ref, ws_ref, ia_ref):
    g = g_ref[...]
    att2 = att_ref[...]
    s = jnp.dot(g, att2, preferred_element_type=jnp.float32)
    ai = s[:, :4]
    aj = s[:, 4:]
    t = ai + aj
    ws = jnp.exp(jnp.where(t > 0, t, 0.2 * t))
    ai_ref[...] = ai
    aj_ref[...] = aj
    ws_ref[...] = ws
    ia_ref[...] = g * jnp.repeat(ws, OUT_CH, axis=1)


def _stage4_body(num_ref, den_ref, bias_ref, out_ref):
    u = num_ref[...] / jnp.maximum(den_ref[...], 1e-16) + bias_ref[...][None, :]
    u = jnp.maximum(u, 0.0)
    un = _rownorm(u)
    y = jnp.tanh(un) * u / un
    out_ref[...] = _proj_rows(y)


def _tc_stage1(x, wt, b):
    return pl.pallas_call(
        _stage1_body,
        grid=(N // BLK,),
        in_specs=[
            pl.BlockSpec((BLK, D), lambda i: (i, 0)),
            pl.BlockSpec((D, D), lambda i: (0, 0)),
            pl.BlockSpec((D,), lambda i: (0,)),
        ],
        out_specs=pl.BlockSpec((BLK, D), lambda i: (i, 0)),
        out_shape=jax.ShapeDtypeStruct((N, D), jnp.float32),
    )(x, wt, b)


def _tc_stage2(g, att2):
    return pl.pallas_call(
        _stage2_body,
        grid=(N // BLK,),
        in_specs=[
            pl.BlockSpec((BLK, D), lambda i: (i, 0)),
            pl.BlockSpec((D, 2 * HEADS), lambda i: (0, 0)),
        ],
        out_specs=[
            pl.BlockSpec((BLK, HEADS), lambda i: (i, 0)),
            pl.BlockSpec((BLK, HEADS), lambda i: (i, 0)),
            pl.BlockSpec((BLK, HEADS), lambda i: (i, 0)),
            pl.BlockSpec((BLK, D), lambda i: (i, 0)),
        ],
        out_shape=[
            jax.ShapeDtypeStruct((N, HEADS), jnp.float32),
            jax.ShapeDtypeStruct((N, HEADS), jnp.float32),
            jax.ShapeDtypeStruct((N, HEADS), jnp.float32),
            jax.ShapeDtypeStruct((N, D), jnp.float32),
        ],
    )(g, att2)


def _tc_stage4(num, den, bias):
    return pl.pallas_call(
        _stage4_body,
        grid=(N // BLK,),
        in_specs=[
            pl.BlockSpec((BLK, D), lambda i: (i, 0)),
            pl.BlockSpec((BLK, D), lambda i: (i, 0)),
            pl.BlockSpec((D,), lambda i: (0,)),
        ],
        out_specs=pl.BlockSpec((BLK, D), lambda i: (i, 0)),
        out_shape=jax.ShapeDtypeStruct((N, D), jnp.float32),
    )(num, den, bias)


def kernel(x, edge_index, lin_weight, lin_bias, att, conv_bias):
    log_x = _tc_stage1(x, lin_weight.T, lin_bias)
    g = jnp.concatenate(
        [log_x[2500 * h:2500 * (h + 1)].reshape(N, OUT_CH) for h in range(HEADS)],
        axis=1,
    )
    att_d = att[:, 0, :OUT_CH]
    att_s = att[:, 0, OUT_CH:]
    eye = jnp.eye(HEADS, dtype=jnp.float32)
    blk_d = jnp.einsum("hc,hk->hck", att_d, eye).reshape(D, HEADS)
    blk_s = jnp.einsum("hc,hk->hck", att_s, eye).reshape(D, HEADS)
    att2 = jnp.concatenate([blk_d, blk_s], axis=1)

    ai, aj, ws, init_aggr = _tc_stage2(g, att2)

    src, dst = edge_index[0], edge_index[1]
    valid = src != dst
    t = ai[dst] + aj[src]
    t = jnp.where(t > 0, t, 0.2 * t)
    w = jnp.where(valid[:, None], jnp.exp(t), 0.0)
    den = ws + jax.ops.segment_sum(w, dst, num_segments=N)
    msg = g[src] * jnp.repeat(w, OUT_CH, axis=1)
    num = init_aggr + jax.ops.segment_sum(msg, dst, num_segments=N)

    num_p = jnp.concatenate(
        [num[:, OUT_CH * h:OUT_CH * (h + 1)].reshape(2500, D) for h in range(HEADS)],
        axis=0,
    )
    den_p = jnp.concatenate(
        [jnp.repeat(den[:, h].reshape(2500, HEADS), OUT_CH, axis=1) for h in range(HEADS)],
        axis=0,
    )
    return _tc_stage4(num_p, den_p, conv_bias)


# speedup vs baseline: 37.8313x; 3.1547x over previous
import functools
import jax
import jax.numpy as jnp
from jax import lax
from jax.experimental import pallas as pl
from jax.experimental.pallas import tpu as pltpu
from jax.experimental.pallas import tpu_sc as plsc

C = 1.0
HEADS = 4
OUT_CH = 32
N = 10000
D = 128
E = 320000
MAXN = 1.0 - 4e-3
BLK = 1000

AGW = 144
NPAD = 10240
NW = 32
EPW = E // NW
CH = 80
NCHUNK = EPW // CH
RPT = NPAD // 16


def _artanh(x):
    x = jnp.clip(x, -1 + 1e-7, 1 - 1e-7)
    return 0.5 * jnp.log((1 + x) / (1 - x))


def _rownorm(x):
    return jnp.maximum(jnp.sqrt(jnp.sum(x * x, axis=-1, keepdims=True)), 1e-15)


def _proj_rows(x):
    n = _rownorm(x)
    return jnp.where(n > MAXN, x / n * MAXN, x)


def _stage1_body(x_ref, wt_ref, b_ref, out_ref):
    x = x_ref[...]
    wt = wt_ref[...]
    b = b_ref[...]
    xn = _rownorm(x)
    mx = jnp.dot(x, wt, preferred_element_type=jnp.float32)
    mxn = _rownorm(mx)
    h = jnp.tanh(mxn / xn * _artanh(xn)) * mx / mxn
    h = _proj_rows(h)
    bn = jnp.maximum(jnp.sqrt(jnp.sum(b * b)), 1e-15)
    hb = jnp.tanh(bn) * b / bn
    hbn = jnp.maximum(jnp.sqrt(jnp.sum(hb * hb)), 1e-15)
    hb = jnp.where(hbn > MAXN, hb / hbn * MAXN, hb)
    hb2 = jnp.sum(hb * hb)
    x2 = jnp.sum(h * h, axis=-1, keepdims=True)
    xy = jnp.sum(h * hb[None, :], axis=-1, keepdims=True)
    num = (1 + 2 * xy + hb2) * h + (1 - x2) * hb[None, :]
    den = 1 + 2 * xy + x2 * hb2
    h = num / jnp.maximum(den, 1e-15)
    h = _proj_rows(h)
    hn = _rownorm(h)
    out_ref[...] = h / hn * _artanh(hn)


def _tc_stage1(x, wt, b):
    return pl.pallas_call(
        _stage1_body,
        grid=(N // BLK,),
        in_specs=[
            pl.BlockSpec((BLK, D), lambda i: (i, 0)),
            pl.BlockSpec((D, D), lambda i: (0, 0)),
            pl.BlockSpec((D,), lambda i: (0,)),
        ],
        out_specs=pl.BlockSpec((BLK, D), lambda i: (i, 0)),
        out_shape=jax.ShapeDtypeStruct((N, D), jnp.float32),
    )(x, wt, b)


def _stage2_body(g_ref, att_ref, aij_ref, ws_ref, ia_ref):
    g = g_ref[...]
    att2 = att_ref[...]
    s = jnp.dot(g, att2, preferred_element_type=jnp.float32)
    ai = s[:, :HEADS]
    aj = s[:, HEADS:]
    t = ai + aj
    ws = jnp.exp(jnp.where(t > 0, t, 0.2 * t))
    aij_ref[...] = jnp.concatenate([s, jnp.zeros_like(s)], axis=1)
    ws_ref[...] = ws
    ia_ref[...] = g * jnp.repeat(ws, OUT_CH, axis=1)


def _tc_stage2(g, att2):
    return pl.pallas_call(
        _stage2_body,
        grid=(N // BLK,),
        in_specs=[
            pl.BlockSpec((BLK, D), lambda i: (i, 0)),
            pl.BlockSpec((D, 2 * HEADS), lambda i: (0, 0)),
        ],
        out_specs=[
            pl.BlockSpec((BLK, 16), lambda i: (i, 0)),
            pl.BlockSpec((BLK, HEADS), lambda i: (i, 0)),
            pl.BlockSpec((BLK, D), lambda i: (i, 0)),
        ],
        out_shape=[
            jax.ShapeDtypeStruct((N, 16), jnp.float32),
            jax.ShapeDtypeStruct((N, HEADS), jnp.float32),
            jax.ShapeDtypeStruct((N, D), jnp.float32),
        ],
    )(g, att2)


def _sc_edge_body(src_h, dst_h, aij_h, g_h, out_h,
                  eidx, aig, ajg, wbuf, grows, msg, zb, sem, ag_sh):
    c = lax.axis_index("c")
    s = lax.axis_index("s")
    wid = s * 2 + c
    iota = lax.iota(jnp.int32, 16)

    def _zrow(i, _):
        for k in range(AGW // 16):
            zb[i, pl.ds(16 * k, 16)] = jnp.zeros((16,), jnp.float32)
        return _
    lax.fori_loop(0, 16, _zrow, 0)
    def _zcp(j, _):
        pltpu.sync_copy(zb, ag_sh.at[pl.ds(s * RPT + j * 16, 16)])
        return _
    lax.fori_loop(0, RPT // 16, _zcp, 0)
    def _zpad(e, _):
        msg[e, pl.ds(128, 16)] = jnp.zeros((16,), jnp.float32)
        return _
    lax.fori_loop(0, CH, _zpad, 0)
    plsc.subcore_barrier()

    def _chunk(t, _):
        base_e = pl.multiple_of(wid * EPW + t * CH, 8)
        pltpu.sync_copy(src_h.at[pl.ds(base_e, CH)], eidx.at[0])
        pltpu.sync_copy(dst_h.at[pl.ds(base_e, CH)], eidx.at[1])
        pltpu.async_copy(aij_h.at[eidx.at[1]], aig, sem).wait()
        pltpu.async_copy(aij_h.at[eidx.at[0]], ajg, sem).wait()
        pltpu.async_copy(g_h.at[eidx.at[0]], grows, sem).wait()

        zero16 = jnp.zeros((16,), jnp.int32)
        one16 = zero16 + 1
        for j in range(CH * HEADS // 16):
            fl = iota + 16 * j
            e_ln = lax.shift_right_logical(fl, 2)
            h_ln = lax.bitwise_and(fl, jnp.full((16,), 3, jnp.int32))
            av = plsc.load_gather(aig, [e_ln, h_ln])
            jv = plsc.load_gather(ajg, [e_ln, h_ln + HEADS])
            tt = av + jv
            tt = jnp.maximum(tt, 0.2 * tt)
            w = jnp.exp(tt)
            sv = plsc.load_gather(eidx, [zero16, e_ln])
            dv = plsc.load_gather(eidx, [one16, e_ln])
            w = jnp.where(sv == dv, jnp.zeros((16,), jnp.float32), w)
            wbuf[pl.ds(16 * j, 16)] = w

        def _edge(e, _):
            wsp = [plsc.load_gather(wbuf, [jnp.full((16,), 4 * e, jnp.int32) + h])
                   for h in range(HEADS)]
            for r in range(8):
                v = grows[e, pl.ds(16 * r, 16)]
                msg[e, pl.ds(16 * r, 16)] = v * wsp[r // 2]
            return _
        lax.fori_loop(0, CH, _edge, 0)

        for q in range(CH * HEADS // 16):
            e_ln = jnp.full((16,), 4 * q, jnp.int32) + lax.shift_right_logical(iota, 2)
            c_ln = jnp.full((16,), 128, jnp.int32) + lax.bitwise_and(iota, jnp.full((16,), 3, jnp.int32))
            plsc.store_scatter(msg, [e_ln, c_ln], wbuf[pl.ds(16 * q, 16)])

        pltpu.sync_copy(msg, ag_sh.at[eidx.at[1]], add=True)
        return _
    lax.fori_loop(0, NCHUNK, _chunk, 0)

    plsc.subcore_barrier()
    def _rcp(j, _):
        rb = s * RPT + j * 16
        pltpu.sync_copy(ag_sh.at[pl.ds(rb, 16)], out_h.at[c, pl.ds(rb, 16)])
        return _
    lax.fori_loop(0, RPT // 16, _rcp, 0)


def _sc_edge(src, dst, aij, g):
    mesh = plsc.VectorSubcoreMesh(core_axis_name="c", subcore_axis_name="s")
    f = pl.kernel(
        _sc_edge_body,
        mesh=mesh,
        compiler_params=pltpu.CompilerParams(use_tc_tiling_on_sc=False,
                                             needs_layout_passes=False),
        out_type=jax.ShapeDtypeStruct((2, NPAD, AGW), jnp.float32),
        scratch_types=[
            pltpu.VMEM((2, CH), jnp.int32),
            pltpu.VMEM((CH, 16), jnp.float32),
            pltpu.VMEM((CH, 16), jnp.float32),
            pltpu.VMEM((CH * HEADS,), jnp.float32),
            pltpu.VMEM((CH, D), jnp.float32),
            pltpu.VMEM((CH, AGW), jnp.float32),
            pltpu.VMEM((16, AGW), jnp.float32),
            pltpu.SemaphoreType.DMA,
            pltpu.VMEM_SHARED((NPAD, AGW), jnp.float32),
        ],
    )
    return f(src, dst, aij, g)


def _stage4a_body(ag_ref, ia_ref, ws_ref, bias_ref, out_ref):
    ag = ag_ref[...]
    tot = ag[0] + ag[1]
    num = tot[:, :D] + ia_ref[...]
    den = tot[:, D:D + HEADS] + ws_ref[...]
    den = jnp.repeat(den, OUT_CH, axis=1)
    u = num / jnp.maximum(den, 1e-16) + bias_ref[...]
    out_ref[...] = jnp.maximum(u, 0.0)


def _tc_stage4a(ag, ia, ws, bias_tile):
    return pl.pallas_call(
        _stage4a_body,
        grid=(N // BLK,),
        in_specs=[
            pl.BlockSpec((2, BLK, AGW), lambda i: (0, i, 0)),
            pl.BlockSpec((BLK, D), lambda i: (i, 0)),
            pl.BlockSpec((BLK, HEADS), lambda i: (i, 0)),
            pl.BlockSpec((BLK, D), lambda i: (i, 0)),
        ],
        out_specs=pl.BlockSpec((BLK, D), lambda i: (i, 0)),
        out_shape=jax.ShapeDtypeStruct((N, D), jnp.float32),
    )(ag, ia, ws, bias_tile)


def _stage4b_body(u_ref, out_ref):
    u = u_ref[...]
    un = _rownorm(u)
    y = jnp.tanh(un) * u / un
    out_ref[...] = _proj_rows(y)


def _tc_stage4b(u):
    return pl.pallas_call(
        _stage4b_body,
        grid=(N // BLK,),
        in_specs=[pl.BlockSpec((BLK, D), lambda i: (i, 0))],
        out_specs=pl.BlockSpec((BLK, D), lambda i: (i, 0)),
        out_shape=jax.ShapeDtypeStruct((N, D), jnp.float32),
    )(u)


def kernel(x, edge_index, lin_weight, lin_bias, att, conv_bias):
    log_x = _tc_stage1(x, lin_weight.T, lin_bias)
    g = jnp.concatenate(
        [log_x[2500 * h:2500 * (h + 1)].reshape(N, OUT_CH) for h in range(HEADS)],
        axis=1,
    )
    att_d = att[:, 0, :OUT_CH]
    att_s = att[:, 0, OUT_CH:]
    eye = jnp.eye(HEADS, dtype=jnp.float32)
    blk_d = jnp.einsum("hc,hk->hck", att_d, eye).reshape(D, HEADS)
    blk_s = jnp.einsum("hc,hk->hck", att_s, eye).reshape(D, HEADS)
    att2 = jnp.concatenate([blk_d, blk_s], axis=1)

    aij, ws, init_aggr = _tc_stage2(g, att2)

    ag = _sc_edge(edge_index[0], edge_index[1], aij, g)

    bias_tile = jnp.tile(jnp.tile(conv_bias.reshape(HEADS, OUT_CH), (1, HEADS)),
                         (BLK // HEADS, 1))
    u = _tc_stage4a(ag, init_aggr, ws, bias_tile)
    u_p = jnp.concatenate(
        [u[:, OUT_CH * h:OUT_CH * (h + 1)].reshape(2500, D) for h in range(HEADS)],
        axis=0,
    )
    return _tc_stage4b(u_p)
